# baseline (speedup 1.0000x reference)
import functools

import jax
import jax.numpy as jnp
from jax import lax
from jax.experimental import pallas as pl
from jax.experimental.pallas import tpu as pltpu
from jax.experimental.pallas import tpu_sc as plsc

ALPHA = 5.992277830325989
BETA = -406274.63784969115

NC = 2
NS = 16
L = 16


def _mlp_body(xa_ref, xb_ref, w1_ref, b1_ref, w2_ref, b2_ref, w3_ref, b3_ref,
              out_ref):
    half = xa_ref.shape[0]

    def run(x_ref, lo):
        x = x_ref[...].astype(jnp.bfloat16)
        h = lax.dot_general(x, w1_ref[...], (((1,), (0,)), ((), ())),
                            preferred_element_type=jnp.float32)
        h = jnp.maximum(h + b1_ref[...], 0.0).astype(jnp.bfloat16)
        h = lax.dot_general(h, w2_ref[...], (((1,), (0,)), ((), ())),
                            preferred_element_type=jnp.float32)
        h = jnp.maximum(h + b2_ref[...], 0.0).astype(jnp.bfloat16)
        s = lax.dot_general(w3_ref[...], h, (((1,), (1,)), ((), ())),
                            preferred_element_type=jnp.float32)
        out_ref[0, 0, pl.ds(lo, half)] = (s + b3_ref[0, 0])[0]

    run(xa_ref, 0)
    run(xb_ref, half)


def _atom_scalars(atom_node, W1, b1, W2, b2, W3, b3, blk, blk_off, nb):
    n, d = atom_node.shape
    grid = (nb,)
    out = pl.pallas_call(
        _mlp_body,
        grid=grid,
        in_specs=[
            pl.BlockSpec((blk // 2, d), lambda i: (2 * (i + blk_off), 0)),
            pl.BlockSpec((blk // 2, d), lambda i: (2 * (i + blk_off) + 1, 0)),
            pl.BlockSpec((d, d), lambda i: (0, 0)),
            pl.BlockSpec((1, d), lambda i: (0, 0)),
            pl.BlockSpec((d, d), lambda i: (0, 0)),
            pl.BlockSpec((1, d), lambda i: (0, 0)),
            pl.BlockSpec((1, d), lambda i: (0, 0)),
            pl.BlockSpec((1, 1), lambda i: (0, 0)),
        ],
        out_specs=pl.BlockSpec((1, 1, blk), lambda i: (i, 0, 0)),
        out_shape=jax.ShapeDtypeStruct((nb, 1, blk), jnp.float32),
        compiler_params=pltpu.CompilerParams(
            dimension_semantics=("parallel",)),
    )(atom_node, atom_node,
      W1.astype(jnp.bfloat16), b1.reshape(1, d),
      W2.astype(jnp.bfloat16), b2.reshape(1, d),
      W3.reshape(1, d).astype(jnp.bfloat16), b3.reshape(1, 1))
    return out.reshape(nb * blk)


def _segment_sum_sc(vals, batch_i32, seg_idx, acc_zeros, s_out, beta):
    n = vals.shape[0]
    nw = NC * NS
    ch = n // nw
    cv = ch // L
    n_idx_rows = s_out // 128

    mesh = plsc.VectorSubcoreMesh(core_axis_name="c", subcore_axis_name="s")

    @functools.partial(
        pl.kernel,
        out_type=jax.ShapeDtypeStruct((NC, s_out), jnp.float32),
        mesh=mesh,
        compiler_params=pltpu.CompilerParams(needs_layout_passes=False),
        scratch_types=[
            pltpu.VMEM((ch,), jnp.float32),
            pltpu.VMEM((ch,), jnp.int32),
            pltpu.VMEM((L, s_out), jnp.float32),
            pltpu.VMEM((s_out,), jnp.float32),
            pltpu.VMEM((n_idx_rows, 128), jnp.int32),
            pltpu.VMEM_SHARED((s_out,), jnp.float32),
        ],
    )
    def seg_kernel(vals_hbm, batch_hbm, segidx_hbm, zeros_hbm, out_hbm,
                   vals_v, ix_v, acc_v, part_v, idx_v, shared_ref):
        c = lax.axis_index("c")
        s = lax.axis_index("s")
        wid = c * NS + s
        base = wid * ch

        pltpu.sync_copy(vals_hbm.at[pl.ds(base, ch)], vals_v)
        pltpu.sync_copy(batch_hbm.at[pl.ds(base, ch)], ix_v)
        pltpu.sync_copy(segidx_hbm, idx_v)
        pltpu.sync_copy(zeros_hbm, acc_v)

        @pl.when(s == 0)
        def _():
            def zb(j, _):
                part_v[pl.ds(j * L, L)] = jnp.zeros((L,), jnp.float32)
                return 0
            lax.fori_loop(0, s_out // L, zb, 0)
            pltpu.sync_copy(part_v, shared_ref)

        plsc.subcore_barrier()

        lane_row = lax.iota(jnp.int32, L)

        def scatter_step(i, _):
            v = vals_v[pl.ds(i * L, L)]
            ix = ix_v[pl.ds(i * L, L)]
            plsc.addupdate_scatter(acc_v, [lane_row, ix], v)
            return 0
        lax.fori_loop(0, cv, scatter_step, 0)

        def fold_step(j, _):
            acc = jnp.zeros((L,), jnp.float32)
            for r in range(L):
                acc = acc + acc_v[r, pl.ds(j * L, L)]
            part_v[pl.ds(j * L, L)] = acc
            return 0
        lax.fori_loop(0, s_out // L, fold_step, 0)

        for j in range(n_idx_rows):
            pltpu.sync_copy(part_v.at[pl.ds(j * 128, 128)],
                            shared_ref.at[idx_v.at[j]], add=True)

        plsc.subcore_barrier()

        @pl.when(jnp.logical_and(s == 0, c == 0))
        def _():
            pltpu.sync_copy(shared_ref, part_v)
            def af(j, _):
                part_v[pl.ds(j * L, L)] = (
                    part_v[pl.ds(j * L, L)] * ALPHA + beta * 0.5)
                return 0
            lax.fori_loop(0, s_out // L, af, 0)
            pltpu.sync_copy(part_v, out_hbm.at[0])

        @pl.when(jnp.logical_and(s == 0, c == 1))
        def _():
            pltpu.sync_copy(shared_ref, part_v)
            def af(j, _):
                part_v[pl.ds(j * L, L)] = (
                    part_v[pl.ds(j * L, L)] * ALPHA + beta * 0.5)
                return 0
            lax.fori_loop(0, s_out // L, af, 0)
            pltpu.sync_copy(part_v, out_hbm.at[1])

    return seg_kernel(vals, batch_i32, seg_idx, acc_zeros)


def kernel(atom_node, batch, W1, b1, W2, b2, W3, b3):
    n, d = atom_node.shape
    s_out = 1024
    blk = 32000
    nb = n // blk
    batch_i32 = batch.astype(jnp.int32)
    seg_idx = jnp.arange(s_out, dtype=jnp.int32).reshape(s_out // 128, 128)
    acc_zeros = jnp.zeros((L, s_out), jnp.float32)
    vals = _atom_scalars(atom_node, W1, b1, W2, b2, W3, b3, blk, 0, nb)
    partials = _segment_sum_sc(vals, batch_i32, seg_idx, acc_zeros, s_out,
                               BETA)
    return partials[0] + partials[1]


# speedup vs baseline: 1.0104x; 1.0104x over previous
import functools

import jax
import jax.numpy as jnp
from jax import lax
from jax.experimental import pallas as pl
from jax.experimental.pallas import tpu as pltpu
from jax.experimental.pallas import tpu_sc as plsc

ALPHA = 5.992277830325989
BETA = -406274.63784969115

NC = 2
NS = 16
L = 16


def _mlp_body(x_ref, w1_ref, b1_ref, w2_ref, b2_ref, w3_ref, b3_ref, out_ref):
    x = x_ref[...].astype(jnp.bfloat16)
    h = lax.dot_general(x, w1_ref[...], (((1,), (0,)), ((), ())),
                        preferred_element_type=jnp.float32)
    h = jnp.maximum(h + b1_ref[...], 0.0).astype(jnp.bfloat16)
    h = lax.dot_general(h, w2_ref[...], (((1,), (0,)), ((), ())),
                        preferred_element_type=jnp.float32)
    h = jnp.maximum(h + b2_ref[...], 0.0).astype(jnp.bfloat16)
    s = lax.dot_general(w3_ref[...], h, (((1,), (1,)), ((), ())),
                        preferred_element_type=jnp.float32)
    out_ref[0] = s + b3_ref[0, 0]


def _atom_scalars(atom_node, W1, b1, W2, b2, W3, b3, blk, blk_off, nb):
    n, d = atom_node.shape
    grid = (nb,)
    out = pl.pallas_call(
        _mlp_body,
        grid=grid,
        in_specs=[
            pl.BlockSpec((blk, d), lambda i: (i + blk_off, 0)),
            pl.BlockSpec((d, d), lambda i: (0, 0)),
            pl.BlockSpec((1, d), lambda i: (0, 0)),
            pl.BlockSpec((d, d), lambda i: (0, 0)),
            pl.BlockSpec((1, d), lambda i: (0, 0)),
            pl.BlockSpec((1, d), lambda i: (0, 0)),
            pl.BlockSpec((1, 1), lambda i: (0, 0)),
        ],
        out_specs=pl.BlockSpec((1, 1, blk), lambda i: (i, 0, 0)),
        out_shape=jax.ShapeDtypeStruct((nb, 1, blk), jnp.float32),
        compiler_params=pltpu.CompilerParams(
            dimension_semantics=("parallel",)),
    )(atom_node,
      W1.astype(jnp.bfloat16), b1.reshape(1, d),
      W2.astype(jnp.bfloat16), b2.reshape(1, d),
      W3.reshape(1, d).astype(jnp.bfloat16), b3.reshape(1, 1))
    return out.reshape(nb * blk)


def _segment_sum_sc(vals, batch_i32, seg_idx, acc_zeros, s_out, beta):
    n = vals.shape[0]
    nw = NC * NS
    ch = n // nw
    cv = ch // L
    n_idx_rows = s_out // 128

    mesh = plsc.VectorSubcoreMesh(core_axis_name="c", subcore_axis_name="s")

    @functools.partial(
        pl.kernel,
        out_type=jax.ShapeDtypeStruct((NC, s_out), jnp.float32),
        mesh=mesh,
        compiler_params=pltpu.CompilerParams(needs_layout_passes=False),
        scratch_types=[
            pltpu.VMEM((ch,), jnp.float32),
            pltpu.VMEM((ch,), jnp.int32),
            pltpu.VMEM((L, s_out), jnp.float32),
            pltpu.VMEM((s_out,), jnp.float32),
            pltpu.VMEM((n_idx_rows, 128), jnp.int32),
            pltpu.VMEM_SHARED((s_out,), jnp.float32),
        ],
    )
    def seg_kernel(vals_hbm, batch_hbm, segidx_hbm, zeros_hbm, out_hbm,
                   vals_v, ix_v, acc_v, part_v, idx_v, shared_ref):
        c = lax.axis_index("c")
        s = lax.axis_index("s")
        wid = c * NS + s
        base = wid * ch

        pltpu.sync_copy(vals_hbm.at[pl.ds(base, ch)], vals_v)
        pltpu.sync_copy(batch_hbm.at[pl.ds(base, ch)], ix_v)
        pltpu.sync_copy(segidx_hbm, idx_v)
        pltpu.sync_copy(zeros_hbm, acc_v)

        @pl.when(s == 0)
        def _():
            def zb(j, _):
                part_v[pl.ds(j * L, L)] = jnp.zeros((L,), jnp.float32)
                return 0
            lax.fori_loop(0, s_out // L, zb, 0)
            pltpu.sync_copy(part_v, shared_ref)

        plsc.subcore_barrier()

        lane_row = lax.iota(jnp.int32, L)

        unroll = 5
        while cv % unroll:
            unroll -= 1

        def scatter_step(i, _):
            for u in range(unroll):
                v = vals_v[pl.ds((i * unroll + u) * L, L)]
                ix = ix_v[pl.ds((i * unroll + u) * L, L)]
                plsc.addupdate_scatter(acc_v, [lane_row, ix], v)
            return 0
        lax.fori_loop(0, cv // unroll, scatter_step, 0)

        def fold_step(j, _):
            acc = jnp.zeros((L,), jnp.float32)
            for r in range(L):
                acc = acc + acc_v[r, pl.ds(j * L, L)]
            part_v[pl.ds(j * L, L)] = acc
            return 0
        lax.fori_loop(0, s_out // L, fold_step, 0)

        for j in range(n_idx_rows):
            pltpu.sync_copy(part_v.at[pl.ds(j * 128, 128)],
                            shared_ref.at[idx_v.at[j]], add=True)

        plsc.subcore_barrier()

        @pl.when(jnp.logical_and(s == 0, c == 0))
        def _():
            pltpu.sync_copy(shared_ref, part_v)
            def af(j, _):
                part_v[pl.ds(j * L, L)] = (
                    part_v[pl.ds(j * L, L)] * ALPHA + beta * 0.5)
                return 0
            lax.fori_loop(0, s_out // L, af, 0)
            pltpu.sync_copy(part_v, out_hbm.at[0])

        @pl.when(jnp.logical_and(s == 0, c == 1))
        def _():
            pltpu.sync_copy(shared_ref, part_v)
            def af(j, _):
                part_v[pl.ds(j * L, L)] = (
                    part_v[pl.ds(j * L, L)] * ALPHA + beta * 0.5)
                return 0
            lax.fori_loop(0, s_out // L, af, 0)
            pltpu.sync_copy(part_v, out_hbm.at[1])

    return seg_kernel(vals, batch_i32, seg_idx, acc_zeros)


def kernel(atom_node, batch, W1, b1, W2, b2, W3, b3):
    n, d = atom_node.shape
    s_out = 1024
    blk = 32000
    nb = n // blk
    batch_i32 = batch.astype(jnp.int32)
    seg_idx = jnp.arange(s_out, dtype=jnp.int32).reshape(s_out // 128, 128)
    acc_zeros = jnp.zeros((L, s_out), jnp.float32)
    vals = _atom_scalars(atom_node, W1, b1, W2, b2, W3, b3, blk, 0, nb)
    partials = _segment_sum_sc(vals, batch_i32, seg_idx, acc_zeros, s_out,
                               BETA)
    return partials[0] + partials[1]


# speedup vs baseline: 1.0246x; 1.0141x over previous
import functools

import jax
import jax.numpy as jnp
from jax import lax
from jax.experimental import pallas as pl
from jax.experimental.pallas import tpu as pltpu
from jax.experimental.pallas import tpu_sc as plsc

ALPHA = 5.992277830325989
BETA = -406274.63784969115

NC = 2
NS = 16
L = 16


def _mlp_body(x_ref, w1_ref, b1_ref, w2_ref, b2_ref, w3_ref, b3_ref, out_ref):
    x = x_ref[...].astype(jnp.bfloat16)
    h = lax.dot_general(x, w1_ref[...], (((1,), (0,)), ((), ())),
                        preferred_element_type=jnp.float32)
    h = jnp.maximum(h + b1_ref[...], 0.0).astype(jnp.bfloat16)
    h = lax.dot_general(h, w2_ref[...], (((1,), (0,)), ((), ())),
                        preferred_element_type=jnp.float32)
    h = jnp.maximum(h + b2_ref[...], 0.0).astype(jnp.bfloat16)
    s = lax.dot_general(w3_ref[...], h, (((1,), (1,)), ((), ())),
                        preferred_element_type=jnp.float32)
    out_ref[0] = s + b3_ref[0, 0]


def _atom_scalars(atom_node, W1, b1, W2, b2, W3, b3, blk, blk_off, nb):
    n, d = atom_node.shape
    grid = (nb,)
    out = pl.pallas_call(
        _mlp_body,
        grid=grid,
        in_specs=[
            pl.BlockSpec((blk, d), lambda i: (i + blk_off, 0)),
            pl.BlockSpec((d, d), lambda i: (0, 0)),
            pl.BlockSpec((1, d), lambda i: (0, 0)),
            pl.BlockSpec((d, d), lambda i: (0, 0)),
            pl.BlockSpec((1, d), lambda i: (0, 0)),
            pl.BlockSpec((1, d), lambda i: (0, 0)),
            pl.BlockSpec((1, 1), lambda i: (0, 0)),
        ],
        out_specs=pl.BlockSpec((1, 1, blk), lambda i: (i, 0, 0)),
        out_shape=jax.ShapeDtypeStruct((nb, 1, blk), jnp.float32),
        compiler_params=pltpu.CompilerParams(
            dimension_semantics=("parallel",)),
    )(atom_node,
      W1.astype(jnp.bfloat16), b1.reshape(1, d),
      W2.astype(jnp.bfloat16), b2.reshape(1, d),
      W3.reshape(1, d).astype(jnp.bfloat16), b3.reshape(1, 1))
    return out.reshape(nb * blk)


def _segment_sum_sc(vals, batch_i32, seg_idx, acc_zeros, s_out, beta):
    n = vals.shape[0]
    nw = NC * NS
    ch = n // nw
    cv = ch // L
    n_idx_rows = s_out // 128

    mesh = plsc.VectorSubcoreMesh(core_axis_name="c", subcore_axis_name="s")

    @functools.partial(
        pl.kernel,
        out_type=jax.ShapeDtypeStruct((NC, s_out), jnp.float32),
        mesh=mesh,
        compiler_params=pltpu.CompilerParams(needs_layout_passes=False),
        scratch_types=[
            pltpu.VMEM((ch,), jnp.float32),
            pltpu.VMEM((ch,), jnp.int32),
            pltpu.VMEM((L, s_out), jnp.float32),
            pltpu.VMEM((s_out,), jnp.float32),
            pltpu.VMEM((n_idx_rows, 128), jnp.int32),
            pltpu.VMEM_SHARED((s_out,), jnp.float32),
            pltpu.SemaphoreType.DMA,
        ],
    )
    def seg_kernel(vals_hbm, batch_hbm, segidx_hbm, zeros_hbm, out_hbm,
                   vals_v, ix_v, acc_v, part_v, idx_v, shared_ref, sem):
        c = lax.axis_index("c")
        s = lax.axis_index("s")
        wid = c * NS + s
        base = wid * ch

        d1 = pltpu.async_copy(vals_hbm.at[pl.ds(base, ch)], vals_v, sem)
        d2 = pltpu.async_copy(batch_hbm.at[pl.ds(base, ch)], ix_v, sem)
        d3 = pltpu.async_copy(segidx_hbm, idx_v, sem)
        d4 = pltpu.async_copy(zeros_hbm, acc_v, sem)
        d1.wait(); d2.wait(); d3.wait(); d4.wait()

        @pl.when(s == 0)
        def _():
            def zb(j, _):
                part_v[pl.ds(j * L, L)] = jnp.zeros((L,), jnp.float32)
                return 0
            lax.fori_loop(0, s_out // L, zb, 0)
            pltpu.sync_copy(part_v, shared_ref)

        plsc.subcore_barrier()

        lane_row = lax.iota(jnp.int32, L)

        unroll = 5
        while cv % unroll:
            unroll -= 1

        def scatter_step(i, _):
            for u in range(unroll):
                v = vals_v[pl.ds((i * unroll + u) * L, L)]
                ix = ix_v[pl.ds((i * unroll + u) * L, L)]
                plsc.addupdate_scatter(acc_v, [lane_row, ix], v)
            return 0
        lax.fori_loop(0, cv // unroll, scatter_step, 0)

        def fold_step(j, _):
            acc = jnp.zeros((L,), jnp.float32)
            for r in range(L):
                acc = acc + acc_v[r, pl.ds(j * L, L)]
            part_v[pl.ds(j * L, L)] = acc
            return 0
        lax.fori_loop(0, s_out // L, fold_step, 0)

        combines = [
            pltpu.async_copy(part_v.at[pl.ds(j * 128, 128)],
                             shared_ref.at[idx_v.at[j]], sem, add=True)
            for j in range(n_idx_rows)
        ]
        for cpy in combines:
            cpy.wait()

        plsc.subcore_barrier()

        @pl.when(jnp.logical_and(s == 0, c == 0))
        def _():
            pltpu.sync_copy(shared_ref, part_v)
            def af(j, _):
                part_v[pl.ds(j * L, L)] = (
                    part_v[pl.ds(j * L, L)] * ALPHA + beta * 0.5)
                return 0
            lax.fori_loop(0, s_out // L, af, 0)
            pltpu.sync_copy(part_v, out_hbm.at[0])

        @pl.when(jnp.logical_and(s == 0, c == 1))
        def _():
            pltpu.sync_copy(shared_ref, part_v)
            def af(j, _):
                part_v[pl.ds(j * L, L)] = (
                    part_v[pl.ds(j * L, L)] * ALPHA + beta * 0.5)
                return 0
            lax.fori_loop(0, s_out // L, af, 0)
            pltpu.sync_copy(part_v, out_hbm.at[1])

    return seg_kernel(vals, batch_i32, seg_idx, acc_zeros)


def kernel(atom_node, batch, W1, b1, W2, b2, W3, b3):
    n, d = atom_node.shape
    s_out = 1024
    blk = 32000
    nb = n // blk
    batch_i32 = batch.astype(jnp.int32)
    seg_idx = jnp.arange(s_out, dtype=jnp.int32).reshape(s_out // 128, 128)
    acc_zeros = jnp.zeros((L, s_out), jnp.float32)
    vals = _atom_scalars(atom_node, W1, b1, W2, b2, W3, b3, blk, 0, nb)
    partials = _segment_sum_sc(vals, batch_i32, seg_idx, acc_zeros, s_out,
                               BETA)
    return partials[0] + partials[1]
